# baseline (speedup 1.0000x reference)
import jax
import jax.numpy as jnp
from jax.experimental import pallas as pl
from jax.experimental.pallas import tpu as pltpu

K = 1024
D = 64
BETA = 0.25
HW = 1024
B = 16
TILE = 1024


def _vq_kernel(x_ref, emb_ref, q_ref, idx_ref, loss_ref, e2_ref, mat_ref):
    step = pl.program_id(0)
    x = x_ref[0]
    emb = emb_ref[...]

    @pl.when(step == 0)
    def _prep():
        e2_ref[...] = jnp.sum(emb * emb, axis=1, keepdims=True)
        rows = jax.lax.broadcasted_iota(jnp.int32, (8, K), 0)
        kval = jax.lax.broadcasted_iota(jnp.int32, (8, K), 1).astype(jnp.float32)
        mat_ref[...] = jnp.where(rows == 0, 1.0,
                                 jnp.where(rows == 1, kval, 0.0))

    e2 = e2_ref[...]
    x2 = jnp.sum(x * x, axis=0, keepdims=True)
    m = jax.lax.dot_general(
        emb, x, (((1,), (0,)), ((), ())),
        preferred_element_type=jnp.float32,
    )
    dist = (x2 + e2) - 2.0 * m

    minval = jnp.min(dist, axis=0, keepdims=True)
    eqf = (dist == minval).astype(jnp.float32)
    res = jax.lax.dot_general(
        mat_ref[...], eqf, (((1,), (0,)), ((), ())),
        preferred_element_type=jnp.float32,
    )
    cnt = res[0:1]
    has_tie = jnp.max(cnt) > 1.5

    @pl.when(jnp.logical_not(has_tie))
    def _fast():
        quant = jax.lax.dot_general(
            emb, eqf, (((0,), (0,)), ((), ())),
            preferred_element_type=jnp.float32,
        )
        q_ref[0] = quant
        idx_ref[0] = res[1:2].astype(jnp.int32)

    @pl.when(has_tie)
    def _slow():
        idx = jnp.argmin(dist, axis=0)
        kiota = jax.lax.broadcasted_iota(jnp.int32, (K, TILE), 0)
        onehot = (kiota == idx[None, :]).astype(jnp.float32)
        quant = jax.lax.dot_general(
            emb, onehot, (((0,), (0,)), ((), ())),
            preferred_element_type=jnp.float32,
        )
        q_ref[0] = quant
        idx_ref[0] = idx.reshape(1, TILE)

    partial = jnp.sum(minval).reshape(1, 1)

    @pl.when(step == 0)
    def _init():
        loss_ref[...] = partial

    @pl.when(step != 0)
    def _acc():
        loss_ref[...] += partial


@jax.jit
def kernel(enc_pred, embeddings):
    x3 = enc_pred.reshape(B, D, HW)
    q, idx, loss_raw = pl.pallas_call(
        _vq_kernel,
        grid=(B,),
        in_specs=[
            pl.BlockSpec((1, D, TILE), lambda s: (s, 0, 0)),
            pl.BlockSpec((K, D), lambda s: (0, 0)),
        ],
        out_specs=[
            pl.BlockSpec((1, D, TILE), lambda s: (s, 0, 0)),
            pl.BlockSpec((1, 1, TILE), lambda s: (s, 0, 0)),
            pl.BlockSpec((1, 1), lambda s: (0, 0)),
        ],
        out_shape=[
            jax.ShapeDtypeStruct((B, D, TILE), jnp.float32),
            jax.ShapeDtypeStruct((B, 1, TILE), jnp.int32),
            jax.ShapeDtypeStruct((1, 1), jnp.float32),
        ],
        scratch_shapes=[
            pltpu.VMEM((K, 1), jnp.float32),
            pltpu.VMEM((8, K), jnp.float32),
        ],
    )(x3, embeddings)
    quantized_out = q.reshape(B, D, 32, 32)
    indices_out = idx.reshape(B, 1, 32, 32)
    loss = loss_raw[0, 0] * (BETA / (B * HW * D))
    return (quantized_out, loss, indices_out)


# speedup vs baseline: 1.2524x; 1.2524x over previous
import jax
import jax.numpy as jnp
from jax.experimental import pallas as pl
from jax.experimental.pallas import tpu as pltpu

K = 1024
D = 64
BETA = 0.25
HW = 1024
B = 16
BLK_B = 2
GRID = B // BLK_B


def _vq_kernel(x_ref, emb_ref, q_ref, idx_ref, loss_ref, e2_ref):
    step = pl.program_id(0)
    emb = emb_ref[...]

    @pl.when(step == 0)
    def _prep():
        e2_ref[...] = jnp.sum(emb * emb, axis=1, keepdims=True)

    e2 = e2_ref[...]
    partial = jnp.zeros((1, 1), jnp.float32)
    for b in range(BLK_B):
        x = x_ref[b]
        x2 = jnp.sum(x * x, axis=0, keepdims=True)
        m = jax.lax.dot_general(
            emb, x, (((1,), (0,)), ((), ())),
            preferred_element_type=jnp.float32,
        )
        dist = (x2 + e2) - 2.0 * m

        idx = jnp.argmin(dist, axis=0)
        kiota = jax.lax.broadcasted_iota(jnp.int32, (K, HW), 0)
        onehot = (kiota == idx[None, :]).astype(jnp.float32)
        quant = jax.lax.dot_general(
            emb, onehot, (((0,), (0,)), ((), ())),
            preferred_element_type=jnp.float32,
        )

        q_ref[b] = quant
        idx_ref[b] = idx.reshape(1, HW)

        diff = quant - x
        partial = partial + jnp.sum(diff * diff).reshape(1, 1)

    @pl.when(step == 0)
    def _init():
        loss_ref[...] = partial

    @pl.when(step != 0)
    def _acc():
        loss_ref[...] += partial

    @pl.when(step == GRID - 1)
    def _scale():
        loss_ref[...] *= BETA / (B * HW * D)


@jax.jit
def kernel(enc_pred, embeddings):
    x3 = enc_pred.reshape(B, D, HW)
    q, idx, loss = pl.pallas_call(
        _vq_kernel,
        grid=(GRID,),
        in_specs=[
            pl.BlockSpec((BLK_B, D, HW), lambda s: (s, 0, 0)),
            pl.BlockSpec((K, D), lambda s: (0, 0)),
        ],
        out_specs=[
            pl.BlockSpec((BLK_B, D, HW), lambda s: (s, 0, 0)),
            pl.BlockSpec((BLK_B, 1, HW), lambda s: (s, 0, 0)),
            pl.BlockSpec((1, 1), lambda s: (0, 0)),
        ],
        out_shape=[
            jax.ShapeDtypeStruct((B, D, HW), jnp.float32),
            jax.ShapeDtypeStruct((B, 1, HW), jnp.int32),
            jax.ShapeDtypeStruct((1, 1), jnp.float32),
        ],
        scratch_shapes=[pltpu.VMEM((K, 1), jnp.float32)],
    )(x3, embeddings)
    quantized_out = q.reshape(B, D, 32, 32)
    indices_out = idx.reshape(B, 1, 32, 32)
    return (quantized_out, loss.reshape(()), indices_out)


# speedup vs baseline: 1.2838x; 1.0251x over previous
import jax
import jax.numpy as jnp
from jax.experimental import pallas as pl
from jax.experimental.pallas import tpu as pltpu

K = 1024
D = 64
BETA = 0.25
HW = 1024
B = 16
BLK_B = 4
GRID = B // BLK_B


def _vq_kernel(x_ref, emb_ref, q_ref, idx_ref, loss_ref, e2_ref):
    step = pl.program_id(0)
    emb = emb_ref[...]

    @pl.when(step == 0)
    def _prep():
        e2_ref[...] = jnp.sum(emb * emb, axis=1, keepdims=True)

    e2 = e2_ref[...]
    partial = jnp.zeros((1, 1), jnp.float32)
    for b in range(BLK_B):
        x = x_ref[b]
        x2 = jnp.sum(x * x, axis=0, keepdims=True)
        m = jax.lax.dot_general(
            emb, x, (((1,), (0,)), ((), ())),
            preferred_element_type=jnp.float32,
        )
        dist = (x2 + e2) - 2.0 * m

        idx = jnp.argmin(dist, axis=0)
        kiota = jax.lax.broadcasted_iota(jnp.int32, (K, HW), 0)
        onehot = (kiota == idx[None, :]).astype(jnp.float32)
        quant = jax.lax.dot_general(
            emb, onehot, (((0,), (0,)), ((), ())),
            preferred_element_type=jnp.float32,
        )

        q_ref[b] = quant
        idx_ref[b] = idx.reshape(1, HW)

        diff = quant - x
        partial = partial + jnp.sum(diff * diff).reshape(1, 1)

    @pl.when(step == 0)
    def _init():
        loss_ref[...] = partial

    @pl.when(step != 0)
    def _acc():
        loss_ref[...] += partial

    @pl.when(step == GRID - 1)
    def _scale():
        loss_ref[...] *= BETA / (B * HW * D)


@jax.jit
def kernel(enc_pred, embeddings):
    x3 = enc_pred.reshape(B, D, HW)
    q, idx, loss = pl.pallas_call(
        _vq_kernel,
        grid=(GRID,),
        in_specs=[
            pl.BlockSpec((BLK_B, D, HW), lambda s: (s, 0, 0)),
            pl.BlockSpec((K, D), lambda s: (0, 0)),
        ],
        out_specs=[
            pl.BlockSpec((BLK_B, D, HW), lambda s: (s, 0, 0)),
            pl.BlockSpec((BLK_B, 1, HW), lambda s: (s, 0, 0)),
            pl.BlockSpec((1, 1), lambda s: (0, 0)),
        ],
        out_shape=[
            jax.ShapeDtypeStruct((B, D, HW), jnp.float32),
            jax.ShapeDtypeStruct((B, 1, HW), jnp.int32),
            jax.ShapeDtypeStruct((1, 1), jnp.float32),
        ],
        scratch_shapes=[pltpu.VMEM((K, 1), jnp.float32)],
    )(x3, embeddings)
    quantized_out = q.reshape(B, D, 32, 32)
    indices_out = idx.reshape(B, 1, 32, 32)
    return (quantized_out, loss.reshape(()), indices_out)


# speedup vs baseline: 1.2905x; 1.0052x over previous
import jax
import jax.numpy as jnp
from jax.experimental import pallas as pl
from jax.experimental.pallas import tpu as pltpu

K = 1024
D = 64
BETA = 0.25
HW = 1024
B = 16
BLK_B = 8
GRID = B // BLK_B


def _vq_kernel(x_ref, emb_ref, q_ref, idx_ref, loss_ref, e2_ref):
    step = pl.program_id(0)
    emb = emb_ref[...]

    @pl.when(step == 0)
    def _prep():
        e2_ref[...] = jnp.sum(emb * emb, axis=1, keepdims=True)

    e2 = e2_ref[...]
    partial = jnp.zeros((1, 1), jnp.float32)
    for b in range(BLK_B):
        x = x_ref[b]
        x2 = jnp.sum(x * x, axis=0, keepdims=True)
        m = jax.lax.dot_general(
            emb, x, (((1,), (0,)), ((), ())),
            preferred_element_type=jnp.float32,
        )
        dist = (x2 + e2) - 2.0 * m

        idx = jnp.argmin(dist, axis=0)
        kiota = jax.lax.broadcasted_iota(jnp.int32, (K, HW), 0)
        onehot = (kiota == idx[None, :]).astype(jnp.float32)
        quant = jax.lax.dot_general(
            emb, onehot, (((0,), (0,)), ((), ())),
            preferred_element_type=jnp.float32,
        )

        q_ref[b] = quant
        idx_ref[b] = idx.reshape(1, HW)

        diff = quant - x
        partial = partial + jnp.sum(diff * diff).reshape(1, 1)

    @pl.when(step == 0)
    def _init():
        loss_ref[...] = partial

    @pl.when(step != 0)
    def _acc():
        loss_ref[...] += partial

    @pl.when(step == GRID - 1)
    def _scale():
        loss_ref[...] *= BETA / (B * HW * D)


@jax.jit
def kernel(enc_pred, embeddings):
    x3 = enc_pred.reshape(B, D, HW)
    q, idx, loss = pl.pallas_call(
        _vq_kernel,
        grid=(GRID,),
        in_specs=[
            pl.BlockSpec((BLK_B, D, HW), lambda s: (s, 0, 0)),
            pl.BlockSpec((K, D), lambda s: (0, 0)),
        ],
        out_specs=[
            pl.BlockSpec((BLK_B, D, HW), lambda s: (s, 0, 0)),
            pl.BlockSpec((BLK_B, 1, HW), lambda s: (s, 0, 0)),
            pl.BlockSpec((1, 1), lambda s: (0, 0)),
        ],
        out_shape=[
            jax.ShapeDtypeStruct((B, D, HW), jnp.float32),
            jax.ShapeDtypeStruct((B, 1, HW), jnp.int32),
            jax.ShapeDtypeStruct((1, 1), jnp.float32),
        ],
        scratch_shapes=[pltpu.VMEM((K, 1), jnp.float32)],
    )(x3, embeddings)
    quantized_out = q.reshape(B, D, 32, 32)
    indices_out = idx.reshape(B, 1, 32, 32)
    return (quantized_out, loss.reshape(()), indices_out)
